# Initial kernel scaffold; baseline (speedup 1.0000x reference)
import jax
import jax.numpy as jnp
from jax.experimental import pallas as pl


def kernel(props, coords, edge_index, batch, params):
    raise NotImplementedError("write your pallas kernel here")


# speedup vs baseline: 1.1196x; 1.1196x over previous
import functools

import jax
import jax.numpy as jnp
from jax import lax
from jax.experimental import pallas as pl
from jax.experimental.pallas import tpu as pltpu
from jax.experimental.pallas import tpu_sc as plsc

_N = 10000
_E = 320000
_G = 512
_KD = 32
_FF = 32
_HID = 258

_NC = 2
_NS = 16
_NWORK = _NC * _NS
_CHUNK = 128
_CPT = 79
_EPT = _CPT * _CHUNK
_E_PAD = _NWORK * _EPT
_NPT = 632
_N_PAD = _NS * _NPT
_EB = 2048
_NSTEPS = _E_PAD // _EB
_NODE_BLK = 1000
_NB_FINAL = _N // _NODE_BLK


def _tc_call(body, **kw):
    return pl.pallas_call(body, **kw)


def _sc_kernel(**kw):
    return functools.partial(
        pl.kernel,
        compiler_params=pltpu.CompilerParams(use_tc_tiling_on_sc=False),
        **kw)


def _silu(x):
    return x / (1.0 + jnp.exp(-x))


def _lane_sum32(x):
    a = x[:, 0:8] + x[:, 8:16]
    a = a + x[:, 16:24]
    a = a + x[:, 24:32]
    b = a[:, 0:4] + a[:, 4:8]
    c = b[:, 0:2] + b[:, 2:4]
    return c[:, 0:1] + c[:, 1:2]


def _ln(x, g, b):
    mu = _lane_sum32(x) / 32.0
    d = x - mu
    var = _lane_sum32(d * d) / 32.0
    return d / jnp.sqrt(var + 1e-5) * g + b


def _make_mesh():
    return plsc.VectorSubcoreMesh(
        core_axis_name="c", subcore_axis_name="s",
        num_cores=_NC, num_subcores=_NS)


def _gather_pair(table, dst_idx, src_idx, width):
    @_sc_kernel(
        out_type=[jax.ShapeDtypeStruct((_E_PAD, width), jnp.float32),
                  jax.ShapeDtypeStruct((_E_PAD, width), jnp.float32)],
        mesh=_make_mesh(),
        scratch_types=[pltpu.VMEM((_CHUNK,), jnp.int32),
                       pltpu.VMEM((_CHUNK,), jnp.int32),
                       pltpu.VMEM((_CHUNK, width), jnp.float32),
                       pltpu.VMEM((_CHUNK, width), jnp.float32),
                       pltpu.SemaphoreType.DMA])
    def gk(tab, dsti, srci, out_d, out_s, idx_d, idx_s, rows_d, rows_s, sem):
        cid = lax.axis_index("c")
        sid = lax.axis_index("s")
        base = (sid * _NC + cid) * _EPT

        def body(c, carry):
            off = base + c * _CHUNK
            pltpu.sync_copy(dsti.at[pl.ds(off, _CHUNK)], idx_d)
            pltpu.sync_copy(srci.at[pl.ds(off, _CHUNK)], idx_s)
            a = pltpu.async_copy(tab.at[idx_d], rows_d, sem)
            b = pltpu.async_copy(tab.at[idx_s], rows_s, sem)
            a.wait()
            b.wait()
            pltpu.sync_copy(rows_d, out_d.at[pl.ds(off, _CHUNK)])
            pltpu.sync_copy(rows_s, out_s.at[pl.ds(off, _CHUNK)])
            return carry

        lax.fori_loop(0, _CPT, body, 0)

    return gk(table, dst_idx, src_idx)


def _scatter_add(msg, dst_idx, zero_rows, width):
    @_sc_kernel(
        out_type=jax.ShapeDtypeStruct((2 * _N_PAD, width), jnp.float32),
        mesh=_make_mesh(),
        scratch_types=[pltpu.VMEM((_CHUNK,), jnp.int32),
                       pltpu.VMEM((_CHUNK, width), jnp.float32),
                       pltpu.VMEM((_NPT, width), jnp.float32),
                       pltpu.VMEM_SHARED((_N_PAD, width), jnp.float32),
                       pltpu.SemaphoreType.DMA])
    def sk(m, dsti, zrows, acc, idx_v, rows_v, drain_v, shacc, sem):
        cid = lax.axis_index("c")
        sid = lax.axis_index("s")
        base = (sid * _NC + cid) * _EPT
        pltpu.sync_copy(zrows, shacc.at[pl.ds(sid * _NPT, _NPT)])
        plsc.subcore_barrier()

        def body(c, carry):
            off = base + c * _CHUNK
            pltpu.sync_copy(dsti.at[pl.ds(off, _CHUNK)], idx_v)
            pltpu.sync_copy(m.at[pl.ds(off, _CHUNK)], rows_v)
            pltpu.sync_copy(rows_v, shacc.at[idx_v], add=True)
            return carry

        lax.fori_loop(0, _CPT, body, 0)
        plsc.subcore_barrier()
        pltpu.sync_copy(shacc.at[pl.ds(sid * _NPT, _NPT)], drain_v)
        pltpu.sync_copy(drain_v, acc.at[pl.ds(cid * _N_PAD + sid * _NPT, _NPT)])

    return sk(msg, dst_idx, zero_rows)


def _prep_call(props, coords_p, emb_wt, emb_b):
    def body(p_r, c_r, w_r, b_r, out_r):
        f = jnp.dot(p_r[...], w_r[...], preferred_element_type=jnp.float32)
        f = f + b_r[...]
        out_r[...] = jnp.concatenate([f, c_r[...]], axis=-1)

    return _tc_call(
        body,
        out_shape=jax.ShapeDtypeStruct((_N, 48), jnp.float32),
    )(props, coords_p, emb_wt, emb_b)


def _edge_body(first, refs):
    if first:
        (ti_r, tj_r, rdin_r, w1_r, b1_r, w2_r, b2_r, g_r, bt_r, isc_r,
         out_r) = refs
        xi = ti_r[:, :32]
        xj = tj_r[:, :32]
        rd = rdin_r[...]
    else:
        (xi_r, xj_r, rdin_r, w1_r, b1_r, w2_r, b2_r, g_r, bt_r,
         isc_r, out_r) = refs
        xi = xi_r[...]
        xj = xj_r[...]
        rd = rdin_r[...]
    xs = rd * isc_r[...]
    cat = jnp.concatenate([xi, xj, jnp.sin(xs), jnp.cos(xs), rd], axis=-1)
    h1 = jnp.dot(cat, w1_r[...], preferred_element_type=jnp.float32)
    h1 = _silu(h1 + b1_r[...])
    m = jnp.dot(h1, w2_r[...], preferred_element_type=jnp.float32)
    m = _silu(m + b2_r[...])
    m = _ln(m, g_r[...], bt_r[...])
    if first:
        out_r[...] = jnp.concatenate(
            [m, jnp.ones((_EB, 16), jnp.float32)], axis=-1)
    else:
        out_r[...] = m


def _full(shape):
    return pl.BlockSpec(shape, lambda k: (0, 0))


def _edge1_call(ti, tj, rd, ew):
    w1, b1, w2, b2, g, bt, isc = ew
    return _tc_call(
        lambda *rs: _edge_body(True, rs),
        grid=(_NSTEPS,),
        in_specs=[pl.BlockSpec((_EB, 48), lambda k: (k, 0)),
                  pl.BlockSpec((_EB, 48), lambda k: (k, 0)),
                  pl.BlockSpec((_EB, 1), lambda k: (k, 0)),
                  _full((129, _HID)), _full((1, _HID)),
                  _full((_HID, 32)), _full((1, 32)), _full((1, 32)),
                  _full((1, 32)), _full((1, 32))],
        out_specs=pl.BlockSpec((_EB, 48), lambda k: (k, 0)),
        out_shape=jax.ShapeDtypeStruct((_E_PAD, 48), jnp.float32),
    )(ti, tj, rd, w1, b1, w2, b2, g, bt, isc)


def _edge23_call(xi, xj, rd, ew):
    w1, b1, w2, b2, g, bt, isc = ew
    return _tc_call(
        lambda *rs: _edge_body(False, rs),
        grid=(_NSTEPS,),
        in_specs=[pl.BlockSpec((_EB, 32), lambda k: (k, 0)),
                  pl.BlockSpec((_EB, 32), lambda k: (k, 0)),
                  pl.BlockSpec((_EB, 1), lambda k: (k, 0)),
                  _full((129, _HID)), _full((1, _HID)),
                  _full((_HID, 32)), _full((1, 32)), _full((1, 32)),
                  _full((1, 32)), _full((1, 32))],
        out_specs=pl.BlockSpec((_EB, 32), lambda k: (k, 0)),
        out_shape=jax.ShapeDtypeStruct((_E_PAD, 32), jnp.float32),
    )(xi, xj, rd, w1, b1, w2, b2, g, bt, isc)


def _node_body(first, refs):
    if first:
        (acc_r, f_r, w1_r, nb1_r, w2_r, nb2_r, eg_r, eb_r,
         g1_r, b1_r, g2_r, b2_r, out_r, cnt_r) = refs
    else:
        (acc_r, cin_r, f_r, w1_r, nb1_r, w2_r, nb2_r, eg_r, eb_r,
         g1_r, b1_r, g2_r, b2_r, out_r) = refs
    a = acc_r[...]
    s = a[:_N] + a[_N_PAD:_N_PAD + _N]
    if first:
        cnt = s[:, 32:33]
        msum = s[:, :32]
        cnt_r[...] = cnt
    else:
        cnt = cin_r[...]
        msum = s
    m_i = msum / jnp.maximum(cnt, 1.0)
    m_i = _ln(m_i, eg_r[...], eb_r[...])
    f = f_r[...]
    hn = _ln(f, g1_r[...], b1_r[...])
    h = jnp.concatenate([hn, m_i], axis=-1)
    h = jnp.dot(h, w1_r[...], preferred_element_type=jnp.float32)
    h = _silu(h + nb1_r[...])
    h = jnp.dot(h, w2_r[...], preferred_element_type=jnp.float32) + nb2_r[...]
    h = _ln(h, g2_r[...], b2_r[...])
    out_r[...] = f + h


def _node1_call(acc, feats, nw):
    return _tc_call(
        lambda *rs: _node_body(True, rs),
        out_shape=[jax.ShapeDtypeStruct((_N, 32), jnp.float32),
                   jax.ShapeDtypeStruct((_N, 1), jnp.float32)],
    )(acc, feats, *nw)


def _node23_call(acc, cnt, feats, nw):
    return _tc_call(
        lambda *rs: _node_body(False, rs),
        out_shape=jax.ShapeDtypeStruct((_N, 32), jnp.float32),
    )(acc, cnt, feats, *nw)


def _final_call(feats, batch3, mlp_ws, gws):
    w1, b1, w2, b2, w3, b3 = mlp_ws
    gw1, gb1, gw2, gb2, gw3, gb3 = gws

    def body(f_r, bt_r, w1_r, b1_r, w2_r, b2_r, w3_r, b3_r,
             gw1_r, gb1_r, gw2_r, gb2_r, gw3_r, gb3_r,
             out_r, pool_s, gcnt_s):
        k = pl.program_id(0)
        h = f_r[...]
        h = _silu(jnp.dot(h, w1_r[...], preferred_element_type=jnp.float32)
                  + b1_r[...])
        h = _silu(jnp.dot(h, w2_r[...], preferred_element_type=jnp.float32)
                  + b2_r[...])
        h = _silu(jnp.dot(h, w3_r[...], preferred_element_type=jnp.float32)
                  + b3_r[...])
        bt = bt_r[0]
        ids = lax.broadcasted_iota(jnp.int32, (_G, _NODE_BLK), 0)
        oh = (ids == bt).astype(jnp.float32)
        p = jnp.dot(oh, h, preferred_element_type=jnp.float32,
                    precision=lax.Precision.HIGHEST)
        c = jnp.sum(oh, axis=-1, keepdims=True)

        @pl.when(k == 0)
        def _():
            pool_s[...] = p
            gcnt_s[...] = c

        @pl.when(k > 0)
        def _():
            pool_s[...] += p
            gcnt_s[...] += c

        @pl.when(k == _NB_FINAL - 1)
        def _():
            g = pool_s[...] / jnp.maximum(gcnt_s[...], 1.0)
            g = _silu(jnp.dot(g, gw1_r[...],
                              preferred_element_type=jnp.float32) + gb1_r[...])
            g = _silu(jnp.dot(g, gw2_r[...],
                              preferred_element_type=jnp.float32) + gb2_r[...])
            out_r[...] = jnp.dot(
                g, gw3_r[...], preferred_element_type=jnp.float32) + gb3_r[...]

    return _tc_call(
        body,
        grid=(_NB_FINAL,),
        in_specs=[pl.BlockSpec((_NODE_BLK, 32), lambda k: (k, 0)),
                  pl.BlockSpec((1, 1, _NODE_BLK), lambda k: (k, 0, 0)),
                  _full((32, 256)), _full((1, 256)),
                  _full((256, 256)), _full((1, 256)),
                  _full((256, 256)), _full((1, 256)),
                  _full((256, 256)), _full((1, 256)),
                  _full((256, 256)), _full((1, 256)),
                  _full((256, 1)), _full((1, 1))],
        out_specs=pl.BlockSpec((_G, 1), lambda k: (0, 0)),
        out_shape=jax.ShapeDtypeStruct((_G, 1), jnp.float32),
        scratch_shapes=[pltpu.VMEM((_G, 256), jnp.float32),
                        pltpu.VMEM((_G, 1), jnp.float32)],
    )(feats, batch3, w1, b1, w2, b2, w3, b3,
      gw1, gb1, gw2, gb2, gw3, gb3)


def kernel(props, coords, edge_index, batch, params):
    f32 = jnp.float32
    order = jnp.argsort(edge_index[1], stable=True)
    src = edge_index[0][order]
    dst = edge_index[1][order]
    pad = _E_PAD - _E
    src_g = jnp.concatenate([src, jnp.zeros((pad,), jnp.int32)])
    dst_g = jnp.concatenate([dst, jnp.zeros((pad,), jnp.int32)])
    dst_s = jnp.concatenate([dst, jnp.full((pad,), _N, jnp.int32)])
    coords_p = jnp.pad(coords, ((0, 0), (0, 13)))
    zero48 = jnp.zeros((_NPT, 48), f32)
    zero32 = jnp.zeros((_NPT, 32), f32)
    inv_sc = (2.0 ** (-jnp.arange(_FF, dtype=f32)))[None, :]

    def edge_w(lp):
        return (lp["eW1"].T, lp["eb1"][None, :],
                lp["eW2"].T, lp["eb2"][None, :],
                lp["en1_g"][None, :], lp["en1_b"][None, :], inv_sc)

    def node_w(lp):
        return (lp["nW1"].T, lp["nb1"][None, :],
                lp["nW2"].T, lp["nb2"][None, :],
                lp["en1_g"][None, :], lp["en1_b"][None, :],
                lp["nn1_g"][None, :], lp["nn1_b"][None, :],
                lp["nn2_g"][None, :], lp["nn2_b"][None, :])

    layers = params["layers"]
    t0 = _prep_call(props, coords_p, params["embed_w"].T,
                    params["embed_b"][None, :])
    feats = t0[:, :32]

    ti, tj = _gather_pair(t0, dst_g, src_g, 48)
    rel = tj[:, 32:35] - ti[:, 32:35]
    rd = jnp.sum(rel * rel, axis=-1, keepdims=True)
    msgx = _edge1_call(ti, tj, rd, edge_w(layers[0]))
    acc = _scatter_add(msgx, dst_s, zero48, 48)
    feats, cnt = _node1_call(acc, feats, node_w(layers[0]))

    for lp in layers[1:]:
        xi, xj = _gather_pair(feats, dst_g, src_g, 32)
        msg = _edge23_call(xi, xj, rd, edge_w(lp))
        acc = _scatter_add(msg, dst_s, zero32, 32)
        feats = _node23_call(acc, cnt, feats, node_w(lp))

    (mw1, mb1), (mw2, mb2), (mw3, mb3) = params["node_mlps"]
    mlp_ws = (mw1.T, mb1[None, :], mw2.T, mb2[None, :], mw3.T, mb3[None, :])
    (gw1, gb1), (gw2, gb2), (gw3, gb3) = params["graph_mlps"]
    gws = (gw1.T, gb1[None, :], gw2.T, gb2[None, :], gw3.T, gb3[None, :])
    batch3 = batch.reshape(_NB_FINAL, 1, _NODE_BLK)

    return _final_call(feats, batch3, mlp_ws, gws)


# speedup vs baseline: 1.1671x; 1.0425x over previous
import functools

import jax
import jax.numpy as jnp
from jax import lax
from jax.experimental import pallas as pl
from jax.experimental.pallas import tpu as pltpu
from jax.experimental.pallas import tpu_sc as plsc

_N = 10000
_E = 320000
_G = 512
_KD = 32
_FF = 32
_HID = 258

_NC = 2
_NS = 16
_NWORK = _NC * _NS
_CHUNK = 128
_CPT = 79
_EPT = _CPT * _CHUNK
_E_PAD = _NWORK * _EPT
_NPT = 632
_N_PAD = _NS * _NPT
_EB = 2048
_NSTEPS = _E_PAD // _EB
_NODE_BLK = 1000
_NB_FINAL = _N // _NODE_BLK


def _tc_call(body, **kw):
    return pl.pallas_call(body, **kw)


def _sc_kernel(**kw):
    return functools.partial(
        pl.kernel,
        compiler_params=pltpu.CompilerParams(use_tc_tiling_on_sc=False),
        **kw)


def _silu(x):
    return x / (1.0 + jnp.exp(-x))


def _lane_sum32(x):
    a = x[:, 0:8] + x[:, 8:16]
    a = a + x[:, 16:24]
    a = a + x[:, 24:32]
    b = a[:, 0:4] + a[:, 4:8]
    c = b[:, 0:2] + b[:, 2:4]
    return c[:, 0:1] + c[:, 1:2]


def _ln(x, g, b):
    mu = _lane_sum32(x) / 32.0
    d = x - mu
    var = _lane_sum32(d * d) / 32.0
    return d / jnp.sqrt(var + 1e-5) * g + b


def _make_mesh():
    return plsc.VectorSubcoreMesh(
        core_axis_name="c", subcore_axis_name="s",
        num_cores=_NC, num_subcores=_NS)


def _gather_pair(table, dst_idx, src_idx, width):
    @_sc_kernel(
        out_type=[jax.ShapeDtypeStruct((_E_PAD, width), jnp.float32),
                  jax.ShapeDtypeStruct((_E_PAD, width), jnp.float32)],
        mesh=_make_mesh(),
        scratch_types=[pltpu.VMEM((_CHUNK,), jnp.int32),
                       pltpu.VMEM((_CHUNK,), jnp.int32),
                       pltpu.VMEM((_CHUNK,), jnp.int32),
                       pltpu.VMEM((_CHUNK,), jnp.int32),
                       pltpu.VMEM((_CHUNK, width), jnp.float32),
                       pltpu.VMEM((_CHUNK, width), jnp.float32),
                       pltpu.VMEM((_CHUNK, width), jnp.float32),
                       pltpu.VMEM((_CHUNK, width), jnp.float32),
                       pltpu.SemaphoreType.DMA,
                       pltpu.SemaphoreType.DMA])
    def gk(tab, dsti, srci, out_d, out_s,
           idx_d0, idx_s0, idx_d1, idx_s1,
           rows_d0, rows_s0, rows_d1, rows_s1, sem0, sem1):
        cid = lax.axis_index("c")
        sid = lax.axis_index("s")
        base = (sid * _NC + cid) * _EPT
        slots = ((idx_d0, idx_s0, rows_d0, rows_s0, sem0),
                 (idx_d1, idx_s1, rows_d1, rows_s1, sem1))

        def issue(slot, c):
            idx_d, idx_s, rows_d, rows_s, sem = slots[slot]
            off = base + c * _CHUNK
            pltpu.sync_copy(dsti.at[pl.ds(off, _CHUNK)], idx_d)
            pltpu.sync_copy(srci.at[pl.ds(off, _CHUNK)], idx_s)
            pltpu.async_copy(tab.at[idx_d], rows_d, sem)
            pltpu.async_copy(tab.at[idx_s], rows_s, sem)

        def drain(slot, c):
            idx_d, idx_s, rows_d, rows_s, sem = slots[slot]
            off = base + c * _CHUNK
            pltpu.make_async_copy(tab.at[idx_d], rows_d, sem).wait()
            pltpu.make_async_copy(tab.at[idx_s], rows_s, sem).wait()
            pltpu.sync_copy(rows_d, out_d.at[pl.ds(off, _CHUNK)])
            pltpu.sync_copy(rows_s, out_s.at[pl.ds(off, _CHUNK)])

        issue(0, 0)

        def body(g, carry):
            c0 = 2 * g
            c1 = c0 + 1

            @pl.when(c1 < _CPT)
            def _():
                issue(1, c1)

            drain(0, c0)

            @pl.when(c0 + 2 < _CPT)
            def _():
                issue(0, c0 + 2)

            @pl.when(c1 < _CPT)
            def _():
                drain(1, c1)

            return carry

        lax.fori_loop(0, (_CPT + 1) // 2, body, 0)

    return gk(table, dst_idx, src_idx)


def _scatter_add(msg, dst_idx, zero_rows, width):
    @_sc_kernel(
        out_type=jax.ShapeDtypeStruct((2 * _N_PAD, width), jnp.float32),
        mesh=_make_mesh(),
        scratch_types=[pltpu.VMEM((_CHUNK,), jnp.int32),
                       pltpu.VMEM((_CHUNK, width), jnp.float32),
                       pltpu.VMEM((_NPT, width), jnp.float32),
                       pltpu.VMEM_SHARED((_N_PAD, width), jnp.float32),
                       pltpu.SemaphoreType.DMA])
    def sk(m, dsti, zrows, acc, idx_v, rows_v, drain_v, shacc, sem):
        cid = lax.axis_index("c")
        sid = lax.axis_index("s")
        base = (sid * _NC + cid) * _EPT
        pltpu.sync_copy(zrows, shacc.at[pl.ds(sid * _NPT, _NPT)])
        plsc.subcore_barrier()

        def body(c, carry):
            off = base + c * _CHUNK
            pltpu.sync_copy(dsti.at[pl.ds(off, _CHUNK)], idx_v)
            pltpu.sync_copy(m.at[pl.ds(off, _CHUNK)], rows_v)
            pltpu.sync_copy(rows_v, shacc.at[idx_v], add=True)
            return carry

        lax.fori_loop(0, _CPT, body, 0)
        plsc.subcore_barrier()
        pltpu.sync_copy(shacc.at[pl.ds(sid * _NPT, _NPT)], drain_v)
        pltpu.sync_copy(drain_v, acc.at[pl.ds(cid * _N_PAD + sid * _NPT, _NPT)])

    return sk(msg, dst_idx, zero_rows)


def _prep_call(props, coords_p, emb_wt, emb_b):
    def body(p_r, c_r, w_r, b_r, out_r):
        f = jnp.dot(p_r[...], w_r[...], preferred_element_type=jnp.float32)
        f = f + b_r[...]
        out_r[...] = jnp.concatenate([f, c_r[...]], axis=-1)

    return _tc_call(
        body,
        out_shape=jax.ShapeDtypeStruct((_N, 48), jnp.float32),
    )(props, coords_p, emb_wt, emb_b)


def _edge_body(first, refs):
    if first:
        (ti_r, tj_r, rdin_r, w1_r, b1_r, w2_r, b2_r, g_r, bt_r, isc_r,
         out_r) = refs
        xi = ti_r[:, :32]
        xj = tj_r[:, :32]
        rd = rdin_r[...]
    else:
        (xi_r, xj_r, rdin_r, w1_r, b1_r, w2_r, b2_r, g_r, bt_r,
         isc_r, out_r) = refs
        xi = xi_r[...]
        xj = xj_r[...]
        rd = rdin_r[...]
    xs = rd * isc_r[...]
    cat = jnp.concatenate([xi, xj, jnp.sin(xs), jnp.cos(xs), rd], axis=-1)
    h1 = jnp.dot(cat, w1_r[...], preferred_element_type=jnp.float32)
    h1 = _silu(h1 + b1_r[...])
    m = jnp.dot(h1, w2_r[...], preferred_element_type=jnp.float32)
    m = _silu(m + b2_r[...])
    m = _ln(m, g_r[...], bt_r[...])
    if first:
        out_r[...] = jnp.concatenate(
            [m, jnp.ones((_EB, 16), jnp.float32)], axis=-1)
    else:
        out_r[...] = m


def _full(shape):
    return pl.BlockSpec(shape, lambda k: (0, 0))


def _edge1_call(ti, tj, rd, ew):
    w1, b1, w2, b2, g, bt, isc = ew
    return _tc_call(
        lambda *rs: _edge_body(True, rs),
        grid=(_NSTEPS,),
        in_specs=[pl.BlockSpec((_EB, 48), lambda k: (k, 0)),
                  pl.BlockSpec((_EB, 48), lambda k: (k, 0)),
                  pl.BlockSpec((_EB, 1), lambda k: (k, 0)),
                  _full((129, _HID)), _full((1, _HID)),
                  _full((_HID, 32)), _full((1, 32)), _full((1, 32)),
                  _full((1, 32)), _full((1, 32))],
        out_specs=pl.BlockSpec((_EB, 48), lambda k: (k, 0)),
        out_shape=jax.ShapeDtypeStruct((_E_PAD, 48), jnp.float32),
    )(ti, tj, rd, w1, b1, w2, b2, g, bt, isc)


def _edge23_call(xi, xj, rd, ew):
    w1, b1, w2, b2, g, bt, isc = ew
    return _tc_call(
        lambda *rs: _edge_body(False, rs),
        grid=(_NSTEPS,),
        in_specs=[pl.BlockSpec((_EB, 32), lambda k: (k, 0)),
                  pl.BlockSpec((_EB, 32), lambda k: (k, 0)),
                  pl.BlockSpec((_EB, 1), lambda k: (k, 0)),
                  _full((129, _HID)), _full((1, _HID)),
                  _full((_HID, 32)), _full((1, 32)), _full((1, 32)),
                  _full((1, 32)), _full((1, 32))],
        out_specs=pl.BlockSpec((_EB, 32), lambda k: (k, 0)),
        out_shape=jax.ShapeDtypeStruct((_E_PAD, 32), jnp.float32),
    )(xi, xj, rd, w1, b1, w2, b2, g, bt, isc)


def _node_body(first, refs):
    if first:
        (acc_r, f_r, w1_r, nb1_r, w2_r, nb2_r, eg_r, eb_r,
         g1_r, b1_r, g2_r, b2_r, out_r, cnt_r) = refs
    else:
        (acc_r, cin_r, f_r, w1_r, nb1_r, w2_r, nb2_r, eg_r, eb_r,
         g1_r, b1_r, g2_r, b2_r, out_r) = refs
    a = acc_r[...]
    s = a[:_N] + a[_N_PAD:_N_PAD + _N]
    if first:
        cnt = s[:, 32:33]
        msum = s[:, :32]
        cnt_r[...] = cnt
    else:
        cnt = cin_r[...]
        msum = s
    m_i = msum / jnp.maximum(cnt, 1.0)
    m_i = _ln(m_i, eg_r[...], eb_r[...])
    f = f_r[...]
    hn = _ln(f, g1_r[...], b1_r[...])
    h = jnp.concatenate([hn, m_i], axis=-1)
    h = jnp.dot(h, w1_r[...], preferred_element_type=jnp.float32)
    h = _silu(h + nb1_r[...])
    h = jnp.dot(h, w2_r[...], preferred_element_type=jnp.float32) + nb2_r[...]
    h = _ln(h, g2_r[...], b2_r[...])
    out_r[...] = f + h


def _node1_call(acc, feats, nw):
    return _tc_call(
        lambda *rs: _node_body(True, rs),
        out_shape=[jax.ShapeDtypeStruct((_N, 32), jnp.float32),
                   jax.ShapeDtypeStruct((_N, 1), jnp.float32)],
    )(acc, feats, *nw)


def _node23_call(acc, cnt, feats, nw):
    return _tc_call(
        lambda *rs: _node_body(False, rs),
        out_shape=jax.ShapeDtypeStruct((_N, 32), jnp.float32),
    )(acc, cnt, feats, *nw)


def _final_call(feats, batch3, mlp_ws, gws):
    w1, b1, w2, b2, w3, b3 = mlp_ws
    gw1, gb1, gw2, gb2, gw3, gb3 = gws

    def body(f_r, bt_r, w1_r, b1_r, w2_r, b2_r, w3_r, b3_r,
             gw1_r, gb1_r, gw2_r, gb2_r, gw3_r, gb3_r,
             out_r, pool_s, gcnt_s):
        k = pl.program_id(0)
        h = f_r[...]
        h = _silu(jnp.dot(h, w1_r[...], preferred_element_type=jnp.float32)
                  + b1_r[...])
        h = _silu(jnp.dot(h, w2_r[...], preferred_element_type=jnp.float32)
                  + b2_r[...])
        h = _silu(jnp.dot(h, w3_r[...], preferred_element_type=jnp.float32)
                  + b3_r[...])
        bt = bt_r[0]
        ids = lax.broadcasted_iota(jnp.int32, (_G, _NODE_BLK), 0)
        oh = (ids == bt).astype(jnp.float32)
        p = jnp.dot(oh, h, preferred_element_type=jnp.float32,
                    precision=lax.Precision.HIGHEST)
        c = jnp.sum(oh, axis=-1, keepdims=True)

        @pl.when(k == 0)
        def _():
            pool_s[...] = p
            gcnt_s[...] = c

        @pl.when(k > 0)
        def _():
            pool_s[...] += p
            gcnt_s[...] += c

        @pl.when(k == _NB_FINAL - 1)
        def _():
            g = pool_s[...] / jnp.maximum(gcnt_s[...], 1.0)
            g = _silu(jnp.dot(g, gw1_r[...],
                              preferred_element_type=jnp.float32) + gb1_r[...])
            g = _silu(jnp.dot(g, gw2_r[...],
                              preferred_element_type=jnp.float32) + gb2_r[...])
            out_r[...] = jnp.dot(
                g, gw3_r[...], preferred_element_type=jnp.float32) + gb3_r[...]

    return _tc_call(
        body,
        grid=(_NB_FINAL,),
        in_specs=[pl.BlockSpec((_NODE_BLK, 32), lambda k: (k, 0)),
                  pl.BlockSpec((1, 1, _NODE_BLK), lambda k: (k, 0, 0)),
                  _full((32, 256)), _full((1, 256)),
                  _full((256, 256)), _full((1, 256)),
                  _full((256, 256)), _full((1, 256)),
                  _full((256, 256)), _full((1, 256)),
                  _full((256, 256)), _full((1, 256)),
                  _full((256, 1)), _full((1, 1))],
        out_specs=pl.BlockSpec((_G, 1), lambda k: (0, 0)),
        out_shape=jax.ShapeDtypeStruct((_G, 1), jnp.float32),
        scratch_shapes=[pltpu.VMEM((_G, 256), jnp.float32),
                        pltpu.VMEM((_G, 1), jnp.float32)],
    )(feats, batch3, w1, b1, w2, b2, w3, b3,
      gw1, gb1, gw2, gb2, gw3, gb3)


def kernel(props, coords, edge_index, batch, params):
    f32 = jnp.float32
    order = jnp.argsort(edge_index[1], stable=True)
    src = edge_index[0][order]
    dst = edge_index[1][order]
    pad = _E_PAD - _E
    src_g = jnp.concatenate([src, jnp.zeros((pad,), jnp.int32)])
    dst_g = jnp.concatenate([dst, jnp.zeros((pad,), jnp.int32)])
    dst_s = jnp.concatenate([dst, jnp.full((pad,), _N, jnp.int32)])
    coords_p = jnp.pad(coords, ((0, 0), (0, 13)))
    zero48 = jnp.zeros((_NPT, 48), f32)
    zero32 = jnp.zeros((_NPT, 32), f32)
    inv_sc = (2.0 ** (-jnp.arange(_FF, dtype=f32)))[None, :]

    def edge_w(lp):
        return (lp["eW1"].T, lp["eb1"][None, :],
                lp["eW2"].T, lp["eb2"][None, :],
                lp["en1_g"][None, :], lp["en1_b"][None, :], inv_sc)

    def node_w(lp):
        return (lp["nW1"].T, lp["nb1"][None, :],
                lp["nW2"].T, lp["nb2"][None, :],
                lp["en1_g"][None, :], lp["en1_b"][None, :],
                lp["nn1_g"][None, :], lp["nn1_b"][None, :],
                lp["nn2_g"][None, :], lp["nn2_b"][None, :])

    layers = params["layers"]
    t0 = _prep_call(props, coords_p, params["embed_w"].T,
                    params["embed_b"][None, :])
    feats = t0[:, :32]

    ti, tj = _gather_pair(t0, dst_g, src_g, 48)
    rel = tj[:, 32:35] - ti[:, 32:35]
    rd = jnp.sum(rel * rel, axis=-1, keepdims=True)
    msgx = _edge1_call(ti, tj, rd, edge_w(layers[0]))
    acc = _scatter_add(msgx, dst_s, zero48, 48)
    feats, cnt = _node1_call(acc, feats, node_w(layers[0]))

    for lp in layers[1:]:
        xi, xj = _gather_pair(feats, dst_g, src_g, 32)
        msg = _edge23_call(xi, xj, rd, edge_w(lp))
        acc = _scatter_add(msg, dst_s, zero32, 32)
        feats = _node23_call(acc, cnt, feats, node_w(lp))

    (mw1, mb1), (mw2, mb2), (mw3, mb3) = params["node_mlps"]
    mlp_ws = (mw1.T, mb1[None, :], mw2.T, mb2[None, :], mw3.T, mb3[None, :])
    (gw1, gb1), (gw2, gb2), (gw3, gb3) = params["graph_mlps"]
    gws = (gw1.T, gb1[None, :], gw2.T, gb2[None, :], gw3.T, gb3[None, :])
    batch3 = batch.reshape(_NB_FINAL, 1, _NODE_BLK)

    return _final_call(feats, batch3, mlp_ws, gws)
